# baseline (speedup 1.0000x reference)
import dataclasses

import jax
import jax.numpy as jnp
from jax import lax
from jax.experimental import pallas as pl
from jax.experimental.pallas import tpu as pltpu
from jax.experimental.pallas import tpu_sc as plsc

_ROWS = 16384
_COLS = 1024
_BR = 16
_LANES = 16

_mesh = plsc.VectorSubcoreMesh(core_axis_name="c", subcore_axis_name="s")

_cparams = pltpu.CompilerParams()
if "needs_layout_passes" in pltpu.CompilerParams.__dataclass_fields__:
    _cparams = dataclasses.replace(_cparams, needs_layout_passes=False)


def _sc_body(idx_hbm, b0_hbm, b1_hbm, b2_hbm, x1_hbm, x2_hbm, o_hbm,
             idx_vmem, b0_vmem, b1_vmem, b2_vmem, sem):
    pltpu.async_copy(idx_hbm, idx_vmem, sem).wait()
    pltpu.async_copy(b0_hbm, b0_vmem, sem).wait()
    pltpu.async_copy(b1_hbm, b1_vmem, sem).wait()
    pltpu.async_copy(b2_hbm, b2_vmem, sem).wait()
    idxv = idx_vmem[...]
    c0 = plsc.load_gather(b0_vmem, [idxv])
    c1 = plsc.load_gather(b1_vmem, [idxv])
    c2 = plsc.load_gather(b2_vmem, [idxv])

    def block_body(x1_v, x2_v, o_v):
        @pl.loop(0, _BR)
        def _(r):
            @pl.loop(0, _COLS, step=_LANES)
            def _(c):
                v1 = x1_v.at[r, pl.ds(c, _LANES)][...]
                v2 = x2_v.at[r, pl.ds(c, _LANES)][...]
                o_v.at[r, pl.ds(c, _LANES)][...] = c0 + c1 * v1 + c2 * v2

    pltpu.emit_pipeline(
        block_body,
        grid=(_ROWS // _BR,),
        in_specs=[
            pl.BlockSpec((_BR, _COLS), lambda i: (i, 0)),
            pl.BlockSpec((_BR, _COLS), lambda i: (i, 0)),
        ],
        out_specs=[pl.BlockSpec((_BR, _COLS), lambda i: (i, 0))],
        core_axis_name=("c", "s"),
        dimension_semantics=(pltpu.PARALLEL,),
    )(x1_hbm, x2_hbm, o_hbm)


def kernel(hour_idx, x1, x2, b0, b1, b2):
    idx = jnp.full((_LANES,), hour_idx, jnp.int32)
    k = pl.kernel(
        _sc_body,
        out_type=jax.ShapeDtypeStruct((_ROWS, _COLS), jnp.float32),
        mesh=_mesh,
        scratch_types=[
            pltpu.VMEM((_LANES,), jnp.int32),
            pltpu.VMEM((168,), jnp.float32),
            pltpu.VMEM((168,), jnp.float32),
            pltpu.VMEM((168,), jnp.float32),
            pltpu.SemaphoreType.DMA,
        ],
        compiler_params=_cparams,
    )
    return k(idx, b0, b1, b2, x1, x2)


# speedup vs baseline: 1.0369x; 1.0369x over previous
import dataclasses

import jax
import jax.numpy as jnp
from jax import lax
from jax.experimental import pallas as pl
from jax.experimental.pallas import tpu as pltpu
from jax.experimental.pallas import tpu_sc as plsc

_ROWS = 16384
_COLS = 1024
_LANES = 16
_FLAT_BLK = 16384

_mesh = plsc.VectorSubcoreMesh(core_axis_name="c", subcore_axis_name="s")

_cparams = pltpu.CompilerParams()
if "needs_layout_passes" in pltpu.CompilerParams.__dataclass_fields__:
    _cparams = dataclasses.replace(_cparams, needs_layout_passes=False)


def _sc_body(idx_hbm, b0_hbm, b1_hbm, b2_hbm, x1_hbm, x2_hbm, o_hbm,
             idx_vmem, b0_vmem, b1_vmem, b2_vmem, sem):
    pltpu.async_copy(idx_hbm, idx_vmem, sem).wait()
    pltpu.async_copy(b0_hbm, b0_vmem, sem).wait()
    pltpu.async_copy(b1_hbm, b1_vmem, sem).wait()
    pltpu.async_copy(b2_hbm, b2_vmem, sem).wait()
    idxv = idx_vmem[...]
    c0 = plsc.load_gather(b0_vmem, [idxv])
    c1 = plsc.load_gather(b1_vmem, [idxv])
    c2 = plsc.load_gather(b2_vmem, [idxv])

    def block_body(x1_v, x2_v, o_v):
        @plsc.parallel_loop(0, _FLAT_BLK, step=_LANES, unroll=8)
        def _(c):
            v1 = x1_v.at[pl.ds(c, _LANES)][...]
            v2 = x2_v.at[pl.ds(c, _LANES)][...]
            o_v.at[pl.ds(c, _LANES)][...] = c0 + c1 * v1 + c2 * v2

    pltpu.emit_pipeline(
        block_body,
        grid=(_ROWS * _COLS // _FLAT_BLK,),
        in_specs=[
            pl.BlockSpec((_FLAT_BLK,), lambda i: (i,)),
            pl.BlockSpec((_FLAT_BLK,), lambda i: (i,)),
        ],
        out_specs=[pl.BlockSpec((_FLAT_BLK,), lambda i: (i,))],
        core_axis_name=("c", "s"),
        dimension_semantics=(pltpu.PARALLEL,),
    )(x1_hbm, x2_hbm, o_hbm)


def kernel(hour_idx, x1, x2, b0, b1, b2):
    idx = jnp.full((_LANES,), hour_idx, jnp.int32)
    k = pl.kernel(
        _sc_body,
        out_type=jax.ShapeDtypeStruct((_ROWS * _COLS,), jnp.float32),
        mesh=_mesh,
        scratch_types=[
            pltpu.VMEM((_LANES,), jnp.int32),
            pltpu.VMEM((168,), jnp.float32),
            pltpu.VMEM((168,), jnp.float32),
            pltpu.VMEM((168,), jnp.float32),
            pltpu.SemaphoreType.DMA,
        ],
        compiler_params=_cparams,
    )
    out = k(idx, b0, b1, b2, x1.reshape(-1), x2.reshape(-1))
    return out.reshape(_ROWS, _COLS)


# speedup vs baseline: 1.1495x; 1.1086x over previous
import dataclasses

import jax
import jax.numpy as jnp
from jax import lax
from jax.experimental import pallas as pl
from jax.experimental.pallas import tpu as pltpu
from jax.experimental.pallas import tpu_sc as plsc

_ROWS = 16384
_COLS = 1024
_LANES = 16

_SC_ROWS = 3072
_TC_ROWS = _ROWS - _SC_ROWS
_TC_BLK = 1024
_FLAT_BLK = 16384

_mesh = plsc.VectorSubcoreMesh(core_axis_name="c", subcore_axis_name="s")

_cparams = pltpu.CompilerParams()
if "needs_layout_passes" in pltpu.CompilerParams.__dataclass_fields__:
    _cparams = dataclasses.replace(_cparams, needs_layout_passes=False)


def _tc_body(idx_ref, b0_ref, b1_ref, b2_ref, x1_ref, x2_ref, o_ref):
    h = idx_ref[0]
    c0 = b0_ref[h]
    c1 = b1_ref[h]
    c2 = b2_ref[h]
    o_ref[:] = c0 + c1 * x1_ref[:] + c2 * x2_ref[:]


def _tc_part(idx, b0, b1, b2, x1, x2):
    row_off = _SC_ROWS // _TC_BLK
    return pl.pallas_call(
        _tc_body,
        grid=(_TC_ROWS // _TC_BLK,),
        in_specs=[
            pl.BlockSpec(memory_space=pltpu.SMEM),
            pl.BlockSpec(memory_space=pltpu.SMEM),
            pl.BlockSpec(memory_space=pltpu.SMEM),
            pl.BlockSpec(memory_space=pltpu.SMEM),
            pl.BlockSpec((_TC_BLK, _COLS), lambda i: (i + row_off, 0)),
            pl.BlockSpec((_TC_BLK, _COLS), lambda i: (i + row_off, 0)),
        ],
        out_specs=pl.BlockSpec((_TC_BLK, _COLS), lambda i: (i, 0)),
        out_shape=jax.ShapeDtypeStruct((_TC_ROWS, _COLS), jnp.float32),
    )(idx, b0, b1, b2, x1, x2)


def _sc_body(idx_hbm, b0_hbm, b1_hbm, b2_hbm, x1_hbm, x2_hbm, o_hbm,
             idx_vmem, b0_vmem, b1_vmem, b2_vmem, sem):
    pltpu.async_copy(idx_hbm, idx_vmem, sem).wait()
    pltpu.async_copy(b0_hbm, b0_vmem, sem).wait()
    pltpu.async_copy(b1_hbm, b1_vmem, sem).wait()
    pltpu.async_copy(b2_hbm, b2_vmem, sem).wait()
    idxv = idx_vmem[...]
    c0 = plsc.load_gather(b0_vmem, [idxv])
    c1 = plsc.load_gather(b1_vmem, [idxv])
    c2 = plsc.load_gather(b2_vmem, [idxv])

    def block_body(x1_v, x2_v, o_v):
        @plsc.parallel_loop(0, _FLAT_BLK, step=_LANES, unroll=8)
        def _(c):
            v1 = x1_v.at[pl.ds(c, _LANES)][...]
            v2 = x2_v.at[pl.ds(c, _LANES)][...]
            o_v.at[pl.ds(c, _LANES)][...] = c0 + c1 * v1 + c2 * v2

    pltpu.emit_pipeline(
        block_body,
        grid=(_SC_ROWS * _COLS // _FLAT_BLK,),
        in_specs=[
            pl.BlockSpec((_FLAT_BLK,), lambda i: (i,)),
            pl.BlockSpec((_FLAT_BLK,), lambda i: (i,)),
        ],
        out_specs=[pl.BlockSpec((_FLAT_BLK,), lambda i: (i,))],
        core_axis_name=("c", "s"),
        dimension_semantics=(pltpu.PARALLEL,),
    )(x1_hbm, x2_hbm, o_hbm)


def _sc_part(idxv, b0, b1, b2, x1, x2):
    k = pl.kernel(
        _sc_body,
        out_type=jax.ShapeDtypeStruct((_SC_ROWS * _COLS,), jnp.float32),
        mesh=_mesh,
        scratch_types=[
            pltpu.VMEM((_LANES,), jnp.int32),
            pltpu.VMEM((168,), jnp.float32),
            pltpu.VMEM((168,), jnp.float32),
            pltpu.VMEM((168,), jnp.float32),
            pltpu.SemaphoreType.DMA,
        ],
        compiler_params=_cparams,
    )
    out = k(idxv, b0, b1, b2, x1.reshape(-1), x2.reshape(-1))
    return out.reshape(_SC_ROWS, _COLS)


def kernel(hour_idx, x1, x2, b0, b1, b2):
    idx = jnp.asarray(hour_idx, jnp.int32).reshape(1)
    idxv = jnp.full((_LANES,), hour_idx, jnp.int32)
    sc_out = _sc_part(idxv, b0, b1, b2, x1, x2)
    tc_out = _tc_part(idx, b0, b1, b2, x1, x2)
    full = jnp.concatenate([sc_out, tc_out], axis=0)
    return full


# speedup vs baseline: 2.2331x; 1.9426x over previous
import dataclasses

import jax
import jax.numpy as jnp
from jax import lax
from jax.experimental import pallas as pl
from jax.experimental.pallas import tpu as pltpu
from jax.experimental.pallas import tpu_sc as plsc

_ROWS = 16384
_COLS = 1024
_LANES = 16

_SC_ROWS = 3072
_TC_ROWS = _ROWS - _SC_ROWS
_TC_BLK = 1024
_SC_BR = 16

_mesh = plsc.VectorSubcoreMesh(core_axis_name="c", subcore_axis_name="s")

_cparams = pltpu.CompilerParams()
if "needs_layout_passes" in pltpu.CompilerParams.__dataclass_fields__:
    _cparams = dataclasses.replace(_cparams, needs_layout_passes=False)


def _tc_body(idx_ref, b0_ref, b1_ref, b2_ref, x1_ref, x2_ref, o_ref):
    h = idx_ref[0]
    c0 = b0_ref[h]
    c1 = b1_ref[h]
    c2 = b2_ref[h]
    o_ref[:] = c0 + c1 * x1_ref[:] + c2 * x2_ref[:]


def _tc_part(idx, b0, b1, b2, x1, x2):
    row_off = _SC_ROWS // _TC_BLK
    return pl.pallas_call(
        _tc_body,
        grid=(_TC_ROWS // _TC_BLK,),
        in_specs=[
            pl.BlockSpec(memory_space=pltpu.SMEM),
            pl.BlockSpec(memory_space=pltpu.SMEM),
            pl.BlockSpec(memory_space=pltpu.SMEM),
            pl.BlockSpec(memory_space=pltpu.SMEM),
            pl.BlockSpec((_TC_BLK, _COLS), lambda i: (i + row_off, 0)),
            pl.BlockSpec((_TC_BLK, _COLS), lambda i: (i + row_off, 0)),
        ],
        out_specs=pl.BlockSpec((_TC_BLK, _COLS), lambda i: (i, 0)),
        out_shape=jax.ShapeDtypeStruct((_TC_ROWS, _COLS), jnp.float32),
    )(idx, b0, b1, b2, x1, x2)


def _sc_body(idx_hbm, b0_hbm, b1_hbm, b2_hbm, x1_hbm, x2_hbm, o_hbm,
             idx_vmem, b0_vmem, b1_vmem, b2_vmem, sem):
    pltpu.async_copy(idx_hbm, idx_vmem, sem).wait()
    pltpu.async_copy(b0_hbm, b0_vmem, sem).wait()
    pltpu.async_copy(b1_hbm, b1_vmem, sem).wait()
    pltpu.async_copy(b2_hbm, b2_vmem, sem).wait()
    idxv = idx_vmem[...]
    c0 = plsc.load_gather(b0_vmem, [idxv])
    c1 = plsc.load_gather(b1_vmem, [idxv])
    c2 = plsc.load_gather(b2_vmem, [idxv])

    def block_body(x1_v, x2_v, o_v):
        @pl.loop(0, _SC_BR)
        def _(r):
            @plsc.parallel_loop(0, _COLS, step=_LANES, unroll=8)
            def _(c):
                v1 = x1_v.at[r, pl.ds(c, _LANES)][...]
                v2 = x2_v.at[r, pl.ds(c, _LANES)][...]
                o_v.at[r, pl.ds(c, _LANES)][...] = c0 + c1 * v1 + c2 * v2

    pltpu.emit_pipeline(
        block_body,
        grid=(_SC_ROWS // _SC_BR,),
        in_specs=[
            pl.BlockSpec((_SC_BR, _COLS), lambda i: (i, 0)),
            pl.BlockSpec((_SC_BR, _COLS), lambda i: (i, 0)),
        ],
        out_specs=[pl.BlockSpec((_SC_BR, _COLS), lambda i: (i, 0))],
        core_axis_name=("c", "s"),
        dimension_semantics=(pltpu.PARALLEL,),
    )(x1_hbm, x2_hbm, o_hbm)


def _sc_part(idxv, b0, b1, b2, x1, x2):
    k = pl.kernel(
        _sc_body,
        out_type=jax.ShapeDtypeStruct((_SC_ROWS, _COLS), jnp.float32),
        mesh=_mesh,
        scratch_types=[
            pltpu.VMEM((_LANES,), jnp.int32),
            pltpu.VMEM((168,), jnp.float32),
            pltpu.VMEM((168,), jnp.float32),
            pltpu.VMEM((168,), jnp.float32),
            pltpu.SemaphoreType.DMA,
        ],
        compiler_params=_cparams,
    )
    return k(idxv, b0, b1, b2, x1, x2)


def kernel(hour_idx, x1, x2, b0, b1, b2):
    idx = jnp.asarray(hour_idx, jnp.int32).reshape(1)
    idxv = jnp.full((_LANES,), hour_idx, jnp.int32)
    sc_out = _sc_part(idxv, b0, b1, b2, x1, x2)
    tc_out = _tc_part(idx, b0, b1, b2, x1, x2)
    full = jnp.concatenate([sc_out, tc_out], axis=0)
    return full


# speedup vs baseline: 2.9567x; 1.3241x over previous
import dataclasses

import jax
import jax.numpy as jnp
from jax import lax
from jax.experimental import pallas as pl
from jax.experimental.pallas import tpu as pltpu
from jax.experimental.pallas import tpu_sc as plsc

_ROWS = 16384
_COLS = 1024
_LANES = 16

_SC_ROWS = 3072
_TC_ROWS = _ROWS - _SC_ROWS
_TC_BLK = 1024
_SC_BR = 16

_mesh = plsc.VectorSubcoreMesh(core_axis_name="c", subcore_axis_name="s")

_cparams = pltpu.CompilerParams()
if "needs_layout_passes" in pltpu.CompilerParams.__dataclass_fields__:
    _cparams = dataclasses.replace(_cparams, needs_layout_passes=False)


def _tc_body(idx_ref, b0_ref, b1_ref, b2_ref, x1_ref, x2_ref, o_ref):
    h = idx_ref[0]
    c0 = b0_ref[h]
    c1 = b1_ref[h]
    c2 = b2_ref[h]
    o_ref[:] = c0 + c1 * x1_ref[:] + c2 * x2_ref[:]


def _tc_part(idx, b0, b1, b2, x1, x2):
    row_off = _SC_ROWS // _TC_BLK
    return pl.pallas_call(
        _tc_body,
        grid=(_TC_ROWS // _TC_BLK,),
        in_specs=[
            pl.BlockSpec(memory_space=pltpu.SMEM),
            pl.BlockSpec(memory_space=pltpu.SMEM),
            pl.BlockSpec(memory_space=pltpu.SMEM),
            pl.BlockSpec(memory_space=pltpu.SMEM),
            pl.BlockSpec((_TC_BLK, _COLS), lambda i: (i + row_off, 0)),
            pl.BlockSpec((_TC_BLK, _COLS), lambda i: (i + row_off, 0)),
        ],
        out_specs=pl.BlockSpec((_TC_BLK, _COLS), lambda i: (i + row_off, 0)),
        out_shape=jax.ShapeDtypeStruct((_ROWS, _COLS), jnp.float32),
    )(idx, b0, b1, b2, x1, x2)


def _sc_body(idx_hbm, b0_hbm, b1_hbm, b2_hbm, x1_hbm, x2_hbm, o_hbm,
             idx_vmem, b0_vmem, b1_vmem, b2_vmem, sem):
    pltpu.async_copy(idx_hbm, idx_vmem, sem).wait()
    pltpu.async_copy(b0_hbm, b0_vmem, sem).wait()
    pltpu.async_copy(b1_hbm, b1_vmem, sem).wait()
    pltpu.async_copy(b2_hbm, b2_vmem, sem).wait()
    idxv = idx_vmem[...]
    c0 = plsc.load_gather(b0_vmem, [idxv])
    c1 = plsc.load_gather(b1_vmem, [idxv])
    c2 = plsc.load_gather(b2_vmem, [idxv])

    def block_body(x1_v, x2_v, o_v):
        @pl.loop(0, _SC_BR)
        def _(r):
            @plsc.parallel_loop(0, _COLS, step=_LANES, unroll=8)
            def _(c):
                v1 = x1_v.at[r, pl.ds(c, _LANES)][...]
                v2 = x2_v.at[r, pl.ds(c, _LANES)][...]
                o_v.at[r, pl.ds(c, _LANES)][...] = c0 + c1 * v1 + c2 * v2

    pltpu.emit_pipeline(
        block_body,
        grid=(_SC_ROWS // _SC_BR,),
        in_specs=[
            pl.BlockSpec((_SC_BR, _COLS), lambda i: (i, 0)),
            pl.BlockSpec((_SC_BR, _COLS), lambda i: (i, 0)),
        ],
        out_specs=[pl.BlockSpec((_SC_BR, _COLS), lambda i: (i, 0))],
        core_axis_name=("c", "s"),
        dimension_semantics=(pltpu.PARALLEL,),
    )(x1_hbm, x2_hbm, o_hbm)


def _sc_part(idxv, b0, b1, b2, x1, x2):
    k = pl.kernel(
        _sc_body,
        out_type=jax.ShapeDtypeStruct((_SC_ROWS, _COLS), jnp.float32),
        mesh=_mesh,
        scratch_types=[
            pltpu.VMEM((_LANES,), jnp.int32),
            pltpu.VMEM((168,), jnp.float32),
            pltpu.VMEM((168,), jnp.float32),
            pltpu.VMEM((168,), jnp.float32),
            pltpu.SemaphoreType.DMA,
        ],
        compiler_params=_cparams,
    )
    return k(idxv, b0, b1, b2, x1, x2)


def kernel(hour_idx, x1, x2, b0, b1, b2):
    idx = jnp.asarray(hour_idx, jnp.int32).reshape(1)
    idxv = jnp.full((_LANES,), hour_idx, jnp.int32)
    sc_out = _sc_part(idxv, b0, b1, b2, x1, x2)
    tc_out = _tc_part(idx, b0, b1, b2, x1, x2)
    return lax.dynamic_update_slice(tc_out, sc_out, (0, 0))
